# Initial kernel scaffold; baseline (speedup 1.0000x reference)
import jax
import jax.numpy as jnp
from jax.experimental import pallas as pl


def kernel(x, edge_index, W1, b1, W2, b2):
    raise NotImplementedError("write your pallas kernel here")


# speedup vs baseline: 18.3228x; 18.3228x over previous
import functools

import jax
import jax.numpy as jnp
from jax import lax
from jax.experimental import pallas as pl
from jax.experimental.pallas import tpu as pltpu
from jax.experimental.pallas import tpu_sc as plsc

_N = 10000
_E = 320000
_NC = 2
_NS = 16
_NW = _NC * _NS
_EPT = _E // _NW
_K = 80
_NCH = _EPT // _K
_STRIPE = _N // _NS
_ZB = 25
_DEG_D = 16


def _mesh():
    return plsc.VectorSubcoreMesh(core_axis_name="c", subcore_axis_name="s")


_SC_PARAMS = pltpu.CompilerParams(use_tc_tiling_on_sc=False)


def _sc_degree(dst3):
    @functools.partial(
        pl.kernel,
        out_type=jax.ShapeDtypeStruct((_NC, _NS, _STRIPE, _DEG_D), jnp.float32),
        mesh=_mesh(),
        compiler_params=_SC_PARAMS,
        scratch_types=[
            pltpu.VMEM((_NCH, _K), jnp.int32),
            pltpu.VMEM((_K, _DEG_D), jnp.float32),
            pltpu.VMEM((_ZB, _DEG_D), jnp.float32),
            pltpu.VMEM_SHARED((_N, _DEG_D), jnp.float32),
        ],
    )
    def k(dst_hbm, out_hbm, dstv, ones, zbuf, acc):
        c = lax.axis_index("c")
        s = lax.axis_index("s")
        wid = c * _NS + s

        one16 = jnp.ones((16,), jnp.float32)
        zero16 = jnp.zeros((16,), jnp.float32)

        def fill(i, _):
            ones[i] = one16
            return 0

        lax.fori_loop(0, _K, fill, 0)

        def zfill(i, _):
            zbuf[i] = zero16
            return 0

        lax.fori_loop(0, _ZB, zfill, 0)

        base = s * _STRIPE

        def zcopy(i, _):
            pltpu.sync_copy(zbuf, acc.at[pl.ds(base + i * _ZB, _ZB)])
            return 0

        lax.fori_loop(0, _STRIPE // _ZB, zcopy, 0)
        plsc.subcore_barrier()

        pltpu.sync_copy(dst_hbm.at[wid], dstv)

        def chunk(j, _):
            pltpu.sync_copy(ones, acc.at[dstv.at[j]], add=True)
            return 0

        lax.fori_loop(0, _NCH, chunk, 0)
        plsc.subcore_barrier()

        pltpu.sync_copy(acc.at[pl.ds(base, _STRIPE)], out_hbm.at[c, s])

    return k(dst3).reshape(_NC, _N, _DEG_D)


def _sc_segsum(hd, src3, dst3, df):
    @functools.partial(
        pl.kernel,
        out_type=jax.ShapeDtypeStruct((_NC, _NS, _STRIPE, df), jnp.float32),
        mesh=_mesh(),
        compiler_params=_SC_PARAMS,
        scratch_types=[
            pltpu.VMEM((_NCH, _K), jnp.int32),
            pltpu.VMEM((_NCH, _K), jnp.int32),
            pltpu.VMEM((_K, df), jnp.float32),
            pltpu.VMEM((_ZB, df), jnp.float32),
            pltpu.VMEM_SHARED((_N, df), jnp.float32),
            pltpu.SemaphoreType.DMA,
        ],
    )
    def k(hd_hbm, src_hbm, dst_hbm, out_hbm, srcv, dstv, rows, zbuf, acc, sem):
        c = lax.axis_index("c")
        s = lax.axis_index("s")
        wid = c * _NS + s

        zero16 = jnp.zeros((16,), jnp.float32)

        def zfill(i, _):
            for q in range(df // 16):
                zbuf[i, pl.ds(q * 16, 16)] = zero16
            return 0

        lax.fori_loop(0, _ZB, zfill, 0)

        base = s * _STRIPE

        def zcopy(i, _):
            pltpu.sync_copy(zbuf, acc.at[pl.ds(base + i * _ZB, _ZB)])
            return 0

        lax.fori_loop(0, _STRIPE // _ZB, zcopy, 0)
        plsc.subcore_barrier()

        pltpu.sync_copy(src_hbm.at[wid], srcv)
        pltpu.sync_copy(dst_hbm.at[wid], dstv)

        def chunk(j, _):
            pltpu.async_copy(hd_hbm.at[srcv.at[j]], rows, sem).wait()
            pltpu.sync_copy(rows, acc.at[dstv.at[j]], add=True)
            return 0

        lax.fori_loop(0, _NCH, chunk, 0)
        plsc.subcore_barrier()

        pltpu.sync_copy(acc.at[pl.ds(base, _STRIPE)], out_hbm.at[c, s])

    return k(hd, src3, dst3).reshape(_NC, _N, df)


_BM = 400
_GRID = _N // _BM


def _tc_mm1(x, W1, dega, degb):
    d = x.shape[1]

    def body(x_ref, w_ref, da_ref, db_ref, hd_ref, dis_ref):
        deg = da_ref[:, 0:1] + db_ref[:, 0:1] + 1.0
        dis = lax.rsqrt(deg)
        h = jnp.dot(x_ref[...], w_ref[...],
                    preferred_element_type=jnp.float32,
                    precision=lax.Precision.HIGHEST)
        hd_ref[...] = h * dis
        dis_ref[...] = jnp.broadcast_to(dis, (_BM, _DEG_D))

    return pl.pallas_call(
        body,
        grid=(_GRID,),
        in_specs=[
            pl.BlockSpec((_BM, d), lambda i: (i, 0)),
            pl.BlockSpec((d, d), lambda i: (0, 0)),
            pl.BlockSpec((_BM, _DEG_D), lambda i: (i, 0)),
            pl.BlockSpec((_BM, _DEG_D), lambda i: (i, 0)),
        ],
        out_specs=[
            pl.BlockSpec((_BM, d), lambda i: (i, 0)),
            pl.BlockSpec((_BM, _DEG_D), lambda i: (i, 0)),
        ],
        out_shape=[
            jax.ShapeDtypeStruct((_N, d), jnp.float32),
            jax.ShapeDtypeStruct((_N, _DEG_D), jnp.float32),
        ],
    )(x, W1, dega, degb)


def _tc_mid(p1a, p1b, hd1, dis16, b1, W2, c_pad):
    d = hd1.shape[1]
    c_out = W2.shape[1]

    def body(pa_ref, pb_ref, hd_ref, dis_ref, b_ref, w_ref, out_ref):
        dis = dis_ref[:, 0:1]
        pre = (pa_ref[...] + pb_ref[...] + hd_ref[...]) * dis + b_ref[...]
        a1 = jnp.maximum(pre, 0.0)
        h2 = jnp.dot(a1, w_ref[...],
                     preferred_element_type=jnp.float32,
                     precision=lax.Precision.HIGHEST)
        hd2 = h2 * dis
        out_ref[...] = jnp.concatenate(
            [hd2, jnp.zeros((_BM, c_pad - c_out), jnp.float32)], axis=1)

    return pl.pallas_call(
        body,
        grid=(_GRID,),
        in_specs=[
            pl.BlockSpec((_BM, d), lambda i: (i, 0)),
            pl.BlockSpec((_BM, d), lambda i: (i, 0)),
            pl.BlockSpec((_BM, d), lambda i: (i, 0)),
            pl.BlockSpec((_BM, _DEG_D), lambda i: (i, 0)),
            pl.BlockSpec((1, d), lambda i: (0, 0)),
            pl.BlockSpec((d, c_out), lambda i: (0, 0)),
        ],
        out_specs=pl.BlockSpec((_BM, c_pad), lambda i: (i, 0)),
        out_shape=jax.ShapeDtypeStruct((_N, c_pad), jnp.float32),
    )(p1a, p1b, hd1, dis16, b1, W2)


def _tc_out(p2a, p2b, hd2p, dis16, b2):
    c_pad = hd2p.shape[1]
    c_out = b2.shape[1]

    def body(pa_ref, pb_ref, hd_ref, dis_ref, b_ref, out_ref):
        dis = dis_ref[:, 0:1]
        agg = (pa_ref[...] + pb_ref[...] + hd_ref[...])[:, :c_out]
        z = agg * dis + b_ref[...]
        m = jnp.max(z, axis=1, keepdims=True)
        ez = jnp.exp(z - m)
        lse = jnp.log(jnp.sum(ez, axis=1, keepdims=True))
        out_ref[...] = z - m - lse

    return pl.pallas_call(
        body,
        grid=(_GRID,),
        in_specs=[
            pl.BlockSpec((_BM, c_pad), lambda i: (i, 0)),
            pl.BlockSpec((_BM, c_pad), lambda i: (i, 0)),
            pl.BlockSpec((_BM, c_pad), lambda i: (i, 0)),
            pl.BlockSpec((_BM, _DEG_D), lambda i: (i, 0)),
            pl.BlockSpec((1, c_out), lambda i: (0, 0)),
        ],
        out_specs=pl.BlockSpec((_BM, c_out), lambda i: (i, 0)),
        out_shape=jax.ShapeDtypeStruct((_N, c_out), jnp.float32),
    )(p2a, p2b, hd2p, dis16, b2)


def kernel(x, edge_index, W1, b1, W2, b2):
    src3 = edge_index[0].reshape(_NW, _NCH, _K)
    dst3 = edge_index[1].reshape(_NW, _NCH, _K)

    degp = _sc_degree(dst3)
    hd1, dis16 = _tc_mm1(x, W1, degp[0], degp[1])
    p1 = _sc_segsum(hd1, src3, dst3, x.shape[1])
    c_pad = 48
    hd2p = _tc_mid(p1[0], p1[1], hd1, dis16, b1.reshape(1, -1), W2, c_pad)
    p2 = _sc_segsum(hd2p, src3, dst3, c_pad)
    return _tc_out(p2[0], p2[1], hd2p, dis16, b2.reshape(1, -1))


# speedup vs baseline: 26.2198x; 1.4310x over previous
import functools

import jax
import jax.numpy as jnp
from jax import lax
from jax.experimental import pallas as pl
from jax.experimental.pallas import tpu as pltpu
from jax.experimental.pallas import tpu_sc as plsc

_N = 10000
_E = 320000
_NC = 2
_NS = 16
_NW = _NC * _NS
_EPT = _E // _NW
_K = 100
_NCH = _EPT // _K
_NPAIR = _NCH // 2
_STRIPE = _N // _NS
_ZB = 25
_DEG_D = 16


def _mesh():
    return plsc.VectorSubcoreMesh(core_axis_name="c", subcore_axis_name="s")


_SC_PARAMS = pltpu.CompilerParams(use_tc_tiling_on_sc=False)


def _sc_degree(dst3):
    @functools.partial(
        pl.kernel,
        out_type=jax.ShapeDtypeStruct((_NC, _NS, _STRIPE, _DEG_D), jnp.float32),
        mesh=_mesh(),
        compiler_params=_SC_PARAMS,
        scratch_types=[
            pltpu.VMEM((_NCH, _K), jnp.int32),
            pltpu.VMEM((_K, _DEG_D), jnp.float32),
            pltpu.VMEM((_ZB, _DEG_D), jnp.float32),
            pltpu.VMEM_SHARED((_N, _DEG_D), jnp.float32),
        ],
    )
    def k(dst_hbm, out_hbm, dstv, ones, zbuf, acc):
        c = lax.axis_index("c")
        s = lax.axis_index("s")
        wid = c * _NS + s

        one16 = jnp.ones((16,), jnp.float32)
        zero16 = jnp.zeros((16,), jnp.float32)

        def fill(i, _):
            ones[i] = one16
            return 0

        lax.fori_loop(0, _K, fill, 0)

        def zfill(i, _):
            zbuf[i] = zero16
            return 0

        lax.fori_loop(0, _ZB, zfill, 0)

        base = s * _STRIPE

        def zcopy(i, _):
            pltpu.sync_copy(zbuf, acc.at[pl.ds(base + i * _ZB, _ZB)])
            return 0

        lax.fori_loop(0, _STRIPE // _ZB, zcopy, 0)
        plsc.subcore_barrier()

        pltpu.sync_copy(dst_hbm.at[wid], dstv)

        def chunk(j, _):
            pltpu.sync_copy(ones, acc.at[dstv.at[j]], add=True)
            return 0

        lax.fori_loop(0, _NCH, chunk, 0)
        plsc.subcore_barrier()

        pltpu.sync_copy(acc.at[pl.ds(base, _STRIPE)], out_hbm.at[c, s])

    return k(dst3).reshape(_NC, _N, _DEG_D)


def _sc_segsum(hd, src3, dst3, df):
    @functools.partial(
        pl.kernel,
        out_type=jax.ShapeDtypeStruct((_NC, _NS, _STRIPE, df), jnp.float32),
        mesh=_mesh(),
        compiler_params=_SC_PARAMS,
        scratch_types=[
            pltpu.VMEM((_NCH, _K), jnp.int32),
            pltpu.VMEM((_NCH, _K), jnp.int32),
            pltpu.VMEM((_K, df), jnp.float32),
            pltpu.VMEM((_K, df), jnp.float32),
            pltpu.VMEM((_ZB, df), jnp.float32),
            pltpu.VMEM_SHARED((_N, df), jnp.float32),
            pltpu.SemaphoreType.DMA,
            pltpu.SemaphoreType.DMA,
        ],
    )
    def k(hd_hbm, src_hbm, dst_hbm, out_hbm,
          srcv, dstv, rows_a, rows_b, zbuf, acc, sem_a, sem_b):
        c = lax.axis_index("c")
        s = lax.axis_index("s")
        wid = c * _NS + s

        zero16 = jnp.zeros((16,), jnp.float32)

        def zfill(i, _):
            for q in range(df // 16):
                zbuf[i, pl.ds(q * 16, 16)] = zero16
            return 0

        lax.fori_loop(0, _ZB, zfill, 0)

        base = s * _STRIPE

        def zcopy(i, _):
            pltpu.sync_copy(zbuf, acc.at[pl.ds(base + i * _ZB, _ZB)])
            return 0

        lax.fori_loop(0, _STRIPE // _ZB, zcopy, 0)
        plsc.subcore_barrier()

        pltpu.sync_copy(src_hbm.at[wid], srcv)
        pltpu.sync_copy(dst_hbm.at[wid], dstv)

        pltpu.async_copy(hd_hbm.at[srcv.at[0]], rows_a, sem_a)

        def pair(j0, _):
            j = 2 * j0
            pltpu.async_copy(hd_hbm.at[srcv.at[j + 1]], rows_b, sem_b)
            pltpu.make_async_copy(hd_hbm.at[srcv.at[j]], rows_a, sem_a).wait()
            pltpu.sync_copy(rows_a, acc.at[dstv.at[j]], add=True)

            @pl.when(j0 < _NPAIR - 1)
            def _():
                pltpu.async_copy(hd_hbm.at[srcv.at[j + 2]], rows_a, sem_a)

            pltpu.make_async_copy(hd_hbm.at[srcv.at[j + 1]], rows_b, sem_b).wait()
            pltpu.sync_copy(rows_b, acc.at[dstv.at[j + 1]], add=True)
            return 0

        lax.fori_loop(0, _NPAIR, pair, 0)
        plsc.subcore_barrier()

        pltpu.sync_copy(acc.at[pl.ds(base, _STRIPE)], out_hbm.at[c, s])

    return k(hd, src3, dst3).reshape(_NC, _N, df)


_BM = 400
_GRID = _N // _BM


def _tc_mm1(x, W1, dega, degb):
    d = x.shape[1]

    def body(x_ref, w_ref, da_ref, db_ref, hd_ref, dis_ref):
        deg = da_ref[:, 0:1] + db_ref[:, 0:1] + 1.0
        dis = lax.rsqrt(deg)
        h = jnp.dot(x_ref[...], w_ref[...],
                    preferred_element_type=jnp.float32,
                    precision=lax.Precision.HIGHEST)
        hd_ref[...] = h * dis
        dis_ref[...] = jnp.broadcast_to(dis, (_BM, _DEG_D))

    return pl.pallas_call(
        body,
        grid=(_GRID,),
        in_specs=[
            pl.BlockSpec((_BM, d), lambda i: (i, 0)),
            pl.BlockSpec((d, d), lambda i: (0, 0)),
            pl.BlockSpec((_BM, _DEG_D), lambda i: (i, 0)),
            pl.BlockSpec((_BM, _DEG_D), lambda i: (i, 0)),
        ],
        out_specs=[
            pl.BlockSpec((_BM, d), lambda i: (i, 0)),
            pl.BlockSpec((_BM, _DEG_D), lambda i: (i, 0)),
        ],
        out_shape=[
            jax.ShapeDtypeStruct((_N, d), jnp.float32),
            jax.ShapeDtypeStruct((_N, _DEG_D), jnp.float32),
        ],
    )(x, W1, dega, degb)


def _tc_mid(p1a, p1b, hd1, dis16, b1, W2, c_pad):
    d = hd1.shape[1]
    c_out = W2.shape[1]

    def body(pa_ref, pb_ref, hd_ref, dis_ref, b_ref, w_ref, out_ref):
        dis = dis_ref[:, 0:1]
        pre = (pa_ref[...] + pb_ref[...] + hd_ref[...]) * dis + b_ref[...]
        a1 = jnp.maximum(pre, 0.0)
        h2 = jnp.dot(a1, w_ref[...],
                     preferred_element_type=jnp.float32,
                     precision=lax.Precision.HIGHEST)
        hd2 = h2 * dis
        out_ref[...] = jnp.concatenate(
            [hd2, jnp.zeros((_BM, c_pad - c_out), jnp.float32)], axis=1)

    return pl.pallas_call(
        body,
        grid=(_GRID,),
        in_specs=[
            pl.BlockSpec((_BM, d), lambda i: (i, 0)),
            pl.BlockSpec((_BM, d), lambda i: (i, 0)),
            pl.BlockSpec((_BM, d), lambda i: (i, 0)),
            pl.BlockSpec((_BM, _DEG_D), lambda i: (i, 0)),
            pl.BlockSpec((1, d), lambda i: (0, 0)),
            pl.BlockSpec((d, c_out), lambda i: (0, 0)),
        ],
        out_specs=pl.BlockSpec((_BM, c_pad), lambda i: (i, 0)),
        out_shape=jax.ShapeDtypeStruct((_N, c_pad), jnp.float32),
    )(p1a, p1b, hd1, dis16, b1, W2)


def _tc_out(p2a, p2b, hd2p, dis16, b2):
    c_pad = hd2p.shape[1]
    c_out = b2.shape[1]

    def body(pa_ref, pb_ref, hd_ref, dis_ref, b_ref, out_ref):
        dis = dis_ref[:, 0:1]
        agg = (pa_ref[...] + pb_ref[...] + hd_ref[...])[:, :c_out]
        z = agg * dis + b_ref[...]
        m = jnp.max(z, axis=1, keepdims=True)
        ez = jnp.exp(z - m)
        lse = jnp.log(jnp.sum(ez, axis=1, keepdims=True))
        out_ref[...] = z - m - lse

    return pl.pallas_call(
        body,
        grid=(_GRID,),
        in_specs=[
            pl.BlockSpec((_BM, c_pad), lambda i: (i, 0)),
            pl.BlockSpec((_BM, c_pad), lambda i: (i, 0)),
            pl.BlockSpec((_BM, c_pad), lambda i: (i, 0)),
            pl.BlockSpec((_BM, _DEG_D), lambda i: (i, 0)),
            pl.BlockSpec((1, c_out), lambda i: (0, 0)),
        ],
        out_specs=pl.BlockSpec((_BM, c_out), lambda i: (i, 0)),
        out_shape=jax.ShapeDtypeStruct((_N, c_out), jnp.float32),
    )(p2a, p2b, hd2p, dis16, b2)


def kernel(x, edge_index, W1, b1, W2, b2):
    src3 = edge_index[0].reshape(_NW, _NCH, _K)
    dst3 = edge_index[1].reshape(_NW, _NCH, _K)

    degp = _sc_degree(dst3)
    hd1, dis16 = _tc_mm1(x, W1, degp[0], degp[1])
    p1 = _sc_segsum(hd1, src3, dst3, x.shape[1])
    c_pad = 48
    hd2p = _tc_mid(p1[0], p1[1], hd1, dis16, b1.reshape(1, -1), W2, c_pad)
    p2 = _sc_segsum(hd2p, src3, dst3, c_pad)
    return _tc_out(p2[0], p2[1], hd2p, dis16, b2.reshape(1, -1))


# speedup vs baseline: 31.8180x; 1.2135x over previous
import functools

import jax
import jax.numpy as jnp
from jax import lax
from jax.experimental import pallas as pl
from jax.experimental.pallas import tpu as pltpu
from jax.experimental.pallas import tpu_sc as plsc

_N = 10000
_E = 320000
_NC = 2
_NS = 16
_NW = _NC * _NS
_EPT = _E // _NW
_K = 100
_NCH = _EPT // _K
_NPAIR = _NCH // 2
_STRIPE = _N // _NS
_ZB = 25
_DEG_D = 16


def _mesh():
    return plsc.VectorSubcoreMesh(core_axis_name="c", subcore_axis_name="s")


_SC_PARAMS = pltpu.CompilerParams(use_tc_tiling_on_sc=False)


def _sc_degree(edge4):
    @functools.partial(
        pl.kernel,
        out_type=jax.ShapeDtypeStruct((_NC, _N, _DEG_D), jnp.float32),
        mesh=_mesh(),
        compiler_params=_SC_PARAMS,
        scratch_types=[
            pltpu.VMEM((_NCH, _K), jnp.int32),
            pltpu.VMEM((_K, _DEG_D), jnp.float32),
            pltpu.VMEM((_ZB, _DEG_D), jnp.float32),
            pltpu.VMEM_SHARED((_N, _DEG_D), jnp.float32),
        ],
    )
    def k(edge_hbm, out_hbm, dstv, ones, zbuf, acc):
        c = lax.axis_index("c")
        s = lax.axis_index("s")
        wid = c * _NS + s

        one16 = jnp.ones((16,), jnp.float32)
        zero16 = jnp.zeros((16,), jnp.float32)

        def fill(i, _):
            ones[i] = one16
            return 0

        lax.fori_loop(0, _K, fill, 0)

        def zfill(i, _):
            zbuf[i] = zero16
            return 0

        lax.fori_loop(0, _ZB, zfill, 0)

        base = s * _STRIPE

        def zcopy(i, _):
            pltpu.sync_copy(zbuf, acc.at[pl.ds(base + i * _ZB, _ZB)])
            return 0

        lax.fori_loop(0, _STRIPE // _ZB, zcopy, 0)
        plsc.subcore_barrier()

        pltpu.sync_copy(edge_hbm.at[1, wid], dstv)

        def chunk(j, _):
            pltpu.sync_copy(ones, acc.at[dstv.at[j]], add=True)
            return 0

        lax.fori_loop(0, _NCH, chunk, 0)
        plsc.subcore_barrier()

        pltpu.sync_copy(acc.at[pl.ds(base, _STRIPE)],
                        out_hbm.at[c, pl.ds(base, _STRIPE)])

    return k(edge4)


def _sc_segsum(hd, edge4, df):
    @functools.partial(
        pl.kernel,
        out_type=jax.ShapeDtypeStruct((_NC, _N, df), jnp.float32),
        mesh=_mesh(),
        compiler_params=_SC_PARAMS,
        scratch_types=[
            pltpu.VMEM((_NCH, _K), jnp.int32),
            pltpu.VMEM((_NCH, _K), jnp.int32),
            pltpu.VMEM((_K, df), jnp.float32),
            pltpu.VMEM((_K, df), jnp.float32),
            pltpu.VMEM((_ZB, df), jnp.float32),
            pltpu.VMEM_SHARED((_N, df), jnp.float32),
            pltpu.SemaphoreType.DMA,
            pltpu.SemaphoreType.DMA,
        ],
    )
    def k(hd_hbm, edge_hbm, out_hbm,
          srcv, dstv, rows_a, rows_b, zbuf, acc, sem_a, sem_b):
        c = lax.axis_index("c")
        s = lax.axis_index("s")
        wid = c * _NS + s

        zero16 = jnp.zeros((16,), jnp.float32)

        def zfill(i, _):
            for q in range(df // 16):
                zbuf[i, pl.ds(q * 16, 16)] = zero16
            return 0

        lax.fori_loop(0, _ZB, zfill, 0)

        base = s * _STRIPE

        def zcopy(i, _):
            pltpu.sync_copy(zbuf, acc.at[pl.ds(base + i * _ZB, _ZB)])
            return 0

        lax.fori_loop(0, _STRIPE // _ZB, zcopy, 0)
        plsc.subcore_barrier()

        pltpu.sync_copy(edge_hbm.at[0, wid], srcv)
        pltpu.sync_copy(edge_hbm.at[1, wid], dstv)

        pltpu.async_copy(hd_hbm.at[srcv.at[0]], rows_a, sem_a)

        def pair(j0, _):
            j = 2 * j0
            pltpu.async_copy(hd_hbm.at[srcv.at[j + 1]], rows_b, sem_b)
            pltpu.make_async_copy(hd_hbm.at[srcv.at[j]], rows_a, sem_a).wait()
            pltpu.sync_copy(rows_a, acc.at[dstv.at[j]], add=True)

            @pl.when(j0 < _NPAIR - 1)
            def _():
                pltpu.async_copy(hd_hbm.at[srcv.at[j + 2]], rows_a, sem_a)

            pltpu.make_async_copy(hd_hbm.at[srcv.at[j + 1]], rows_b, sem_b).wait()
            pltpu.sync_copy(rows_b, acc.at[dstv.at[j + 1]], add=True)
            return 0

        lax.fori_loop(0, _NPAIR, pair, 0)
        plsc.subcore_barrier()

        pltpu.sync_copy(acc.at[pl.ds(base, _STRIPE)],
                        out_hbm.at[c, pl.ds(base, _STRIPE)])

    return k(hd, edge4)


_BM = 400
_GRID = _N // _BM


def _tc_mm1(x, W1, degp):
    d = x.shape[1]

    def body(x_ref, w_ref, dp_ref, hd_ref, dis_ref):
        deg = dp_ref[0, :, 0:1] + dp_ref[1, :, 0:1] + 1.0
        dis = lax.rsqrt(deg)
        h = jnp.dot(x_ref[...], w_ref[...],
                    preferred_element_type=jnp.float32)
        hd_ref[...] = h * dis
        dis_ref[...] = jnp.broadcast_to(dis, (_BM, _DEG_D))

    return pl.pallas_call(
        body,
        grid=(_GRID,),
        in_specs=[
            pl.BlockSpec((_BM, d), lambda i: (i, 0)),
            pl.BlockSpec((d, d), lambda i: (0, 0)),
            pl.BlockSpec((_NC, _BM, _DEG_D), lambda i: (0, i, 0)),
        ],
        out_specs=[
            pl.BlockSpec((_BM, d), lambda i: (i, 0)),
            pl.BlockSpec((_BM, _DEG_D), lambda i: (i, 0)),
        ],
        out_shape=[
            jax.ShapeDtypeStruct((_N, d), jnp.float32),
            jax.ShapeDtypeStruct((_N, _DEG_D), jnp.float32),
        ],
    )(x, W1, degp)


def _tc_mid(p1, hd1, dis16, b1, W2, c_pad):
    d = hd1.shape[1]
    c_out = W2.shape[1]

    def body(p_ref, hd_ref, dis_ref, b_ref, w_ref, out_ref):
        dis = dis_ref[:, 0:1]
        pre = (p_ref[0] + p_ref[1] + hd_ref[...]) * dis + b_ref[...]
        a1 = jnp.maximum(pre, 0.0)
        h2 = jnp.dot(a1, w_ref[...],
                     preferred_element_type=jnp.float32)
        hd2 = h2 * dis
        out_ref[...] = jnp.concatenate(
            [hd2, jnp.zeros((_BM, c_pad - c_out), jnp.float32)], axis=1)

    return pl.pallas_call(
        body,
        grid=(_GRID,),
        in_specs=[
            pl.BlockSpec((_NC, _BM, d), lambda i: (0, i, 0)),
            pl.BlockSpec((_BM, d), lambda i: (i, 0)),
            pl.BlockSpec((_BM, _DEG_D), lambda i: (i, 0)),
            pl.BlockSpec((1, d), lambda i: (0, 0)),
            pl.BlockSpec((d, c_out), lambda i: (0, 0)),
        ],
        out_specs=pl.BlockSpec((_BM, c_pad), lambda i: (i, 0)),
        out_shape=jax.ShapeDtypeStruct((_N, c_pad), jnp.float32),
    )(p1, hd1, dis16, b1, W2)


def _tc_out(p2, hd2p, dis16, b2):
    c_pad = hd2p.shape[1]
    c_out = b2.shape[1]

    def body(p_ref, hd_ref, dis_ref, b_ref, out_ref):
        dis = dis_ref[:, 0:1]
        agg = (p_ref[0] + p_ref[1] + hd_ref[...])[:, :c_out]
        z = agg * dis + b_ref[...]
        m = jnp.max(z, axis=1, keepdims=True)
        ez = jnp.exp(z - m)
        lse = jnp.log(jnp.sum(ez, axis=1, keepdims=True))
        out_ref[...] = z - m - lse

    return pl.pallas_call(
        body,
        grid=(_GRID,),
        in_specs=[
            pl.BlockSpec((_NC, _BM, c_pad), lambda i: (0, i, 0)),
            pl.BlockSpec((_BM, c_pad), lambda i: (i, 0)),
            pl.BlockSpec((_BM, _DEG_D), lambda i: (i, 0)),
            pl.BlockSpec((1, c_out), lambda i: (0, 0)),
        ],
        out_specs=pl.BlockSpec((_BM, c_out), lambda i: (i, 0)),
        out_shape=jax.ShapeDtypeStruct((_N, c_out), jnp.float32),
    )(p2, hd2p, dis16, b2)


def kernel(x, edge_index, W1, b1, W2, b2):
    edge4 = edge_index.reshape(2, _NW, _NCH, _K)

    degp = _sc_degree(edge4)
    hd1, dis16 = _tc_mm1(x, W1, degp)
    p1 = _sc_segsum(hd1, edge4, x.shape[1])
    c_pad = 48
    hd2p = _tc_mid(p1, hd1, dis16, b1.reshape(1, -1), W2, c_pad)
    p2 = _sc_segsum(hd2p, edge4, c_pad)
    return _tc_out(p2, hd2p, dis16, b2.reshape(1, -1))
